# baseline (speedup 1.0000x reference)
import functools

import jax
import jax.numpy as jnp
from jax import lax
from jax.experimental import pallas as pl
from jax.experimental.pallas import tpu as pltpu
from jax.experimental.pallas import tpu_sc as plsc

_PAD = 1
_ROW_LEN = 8192
_D = 1024
_CHUNK = 1024
_K = 32
_VPT = 512
_NCH = _VPT // _K
_INVW = 8704
_DUMW = 8400
_SEG = _INVW // 8
_BIG = 1 << 30


def _sc_body(inp_hbm, table_hbm, out_hbm,
             row_v, pos_v, val_v, padlist_v, fpall_v, seg_v, gidx_v,
             inv0_v, inv1_v, pidx_v, zidx_v, t0_v, buf0, buf1, buf2,
             inv_sh, meta_sh,
             sg0, sg1, sg2, so0, so1, so2, msem):
    c = lax.axis_index("c")
    s = lax.axis_index("s")
    r_local = s // 8
    p = s % 8
    row_glob = 2 * c + r_local
    flat_row_base = row_glob * _ROW_LEN
    chunk_tok_base = flat_row_base + p * _CHUNK
    chunk_local = p * _CHUNK
    iota = lax.iota(jnp.int32, 16)

    pltpu.sync_copy(inp_hbm.at[pl.ds(pl.multiple_of(row_glob * _ROW_LEN,
                                                    _ROW_LEN), _ROW_LEN)],
                    row_v)

    def mask16(off):
        x = row_v[pl.ds(off, 16)]
        return jnp.minimum(jnp.abs(x - jnp.int32(_PAD)), jnp.int32(1))

    def acc_body(i, acc):
        for j in range(4):
            acc = acc + mask16(i * 64 + j * 16)
        return acc

    acc = lax.fori_loop(0, p * (_CHUNK // 64), acc_body,
                        jnp.zeros((16,), jnp.int32))
    carry = jnp.sum(acc)

    padcnt = jnp.int32(0)
    firstpad = _BIG
    for q in range(8):
        for j in range(8):
            t = q * 128 + j * 16
            m = mask16(chunk_local + t)
            cs = plsc.cumsum(m) + carry
            carry = carry + jnp.sum(m)
            iglob = chunk_tok_base + t + iota
            pos_v[q, pl.ds(j * 16, 16)] = (
                cs * m - 1 + (1 - m) * jnp.int32(_DUMW + 1))
            val_v[pl.ds(t, 16)] = iglob
            pad = 1 - m
            ranks = plsc.cumsum(pad) + padcnt - 1
            lidx = ranks * pad + (1 - pad) * jnp.int32(_CHUNK)
            plsc.store_scatter(padlist_v, [lidx], iglob)
            padcnt = padcnt + jnp.sum(pad)
            firstpad = jnp.minimum(
                firstpad, jnp.min(iglob * pad + m * _BIG))

    pidx_v[pl.ds(0, 16)] = jnp.zeros((16,), jnp.int32) + firstpad
    pltpu.sync_copy(pidx_v, meta_sh.at[pl.ds(pl.multiple_of(s * 16, 16), 16)])
    plsc.subcore_barrier()
    pltpu.sync_copy(meta_sh, fpall_v)
    fpacc = jnp.zeros((16,), jnp.int32) + _BIG
    for t in range(8):
        fpacc = jnp.minimum(
            fpacc, fpall_v[pl.ds(r_local * 128 + t * 16, 16)])
    fp_use = jnp.minimum(jnp.min(fpacc), flat_row_base + _ROW_LEN - 1)

    fpuse_v = jnp.zeros((16,), jnp.int32) + fp_use

    def seg_body(i, cv):
        seg_v[pl.ds(i * 16, 16)] = fpuse_v
        return cv

    lax.fori_loop(0, _SEG // 16, seg_body, 0)
    pltpu.sync_copy(seg_v, inv_sh.at[r_local, pl.ds(p * _SEG, _SEG)])
    plsc.subcore_barrier()

    for q in range(8):
        pltpu.async_copy(val_v.at[pl.ds(q * 128, 128)],
                         inv_sh.at[r_local].at[pos_v.at[q]], msem)
    for q in range(8):
        pltpu.make_async_copy(val_v.at[pl.ds(0, 128)],
                             inv_sh.at[r_local].at[pl.ds(0, 128)],
                             msem).wait()
    plsc.subcore_barrier()

    wbase = s * _VPT
    for g in range(_NCH):
        pltpu.async_copy(inv_sh.at[0, pl.ds(wbase + g * _K, _K)],
                         inv0_v.at[g], msem)
        pltpu.async_copy(inv_sh.at[1, pl.ds(wbase + g * _K, _K)],
                         inv1_v.at[g], msem)
    for g in range(_NCH):
        pltpu.make_async_copy(inv_sh.at[0, pl.ds(0, _K)], inv0_v.at[g],
                              msem).wait()
        pltpu.make_async_copy(inv_sh.at[1, pl.ds(0, _K)], inv1_v.at[g],
                              msem).wait()
    for g in range(_NCH):
        base = wbase + g * _K + 1
        gidx_v[pl.ds(g * _K, 16)] = base + iota
        gidx_v[pl.ds(g * _K + 16, 16)] = base + 16 + iota

    bufs = (buf0, buf1, buf2)
    sgs = (sg0, sg1, sg2)
    sos = (so0, so1, so2)

    def start_g(g, b):
        idx = gidx_v.at[pl.ds(pl.multiple_of(g * _K, _K), _K)]
        pltpu.async_copy(table_hbm.at[idx], bufs[b], sgs[b])

    def wait_g(b):
        pltpu.make_async_copy(table_hbm.at[pl.ds(0, _K)], bufs[b],
                              sgs[b]).wait()

    def start_s(g, b):
        pltpu.async_copy(bufs[b], out_hbm.at[inv0_v.at[g]], sos[b])
        pltpu.async_copy(bufs[b], out_hbm.at[inv1_v.at[g]], sos[b])

    def wait_s(b):
        for _ in range(2):
            pltpu.make_async_copy(bufs[b], out_hbm.at[pl.ds(0, _K)],
                                  sos[b]).wait()

    start_g(0, 0)
    start_g(1, 1)
    wait_g(0)
    start_s(0, 0)
    start_g(2, 2)
    wait_g(1)
    start_s(1, 1)
    wait_s(0)
    start_g(3, 0)

    def ring(i, cv):
        for b in range(3):
            g = 3 * i + 2 + b
            bb = (2 + b) % 3
            wait_g(bb)
            start_s(g, bb)
            wait_s((1 + b) % 3)
            start_g(g + 2, (1 + b) % 3)
        return cv

    lax.fori_loop(0, (_NCH - 4) // 3, ring, 0)
    wait_g(2)
    start_s(14, 2)
    wait_s(1)
    wait_g(0)
    start_s(15, 0)
    wait_s(2)
    wait_s(0)
    plsc.subcore_barrier()

    zidx_v[pl.ds(0, 16)] = jnp.zeros((16,), jnp.int32)
    pltpu.async_copy(table_hbm.at[zidx_v], t0_v, msem).wait()

    def pad_body(k, cv):
        jvec = k * 16 + iota
        plv = padlist_v[pl.ds(pl.multiple_of(k * 16, 16), 16)]
        valid = jnp.minimum(jnp.maximum(padcnt - jvec, 0), 1)
        pidx_v[pl.ds(0, 16)] = plv * valid + (1 - valid) * firstpad
        pltpu.async_copy(t0_v, out_hbm.at[pidx_v], msem).wait()
        return cv

    lax.fori_loop(0, (padcnt + 15) // 16, pad_body, 0)


@jax.jit
def _lpe(flat_inp, table):
    n_tokens = flat_inp.shape[0]
    mesh = plsc.VectorSubcoreMesh(core_axis_name="c", subcore_axis_name="s")
    call = functools.partial(
        pl.kernel,
        mesh=mesh,
        out_type=jax.ShapeDtypeStruct((n_tokens, _D), jnp.float32),
        scratch_types=[
            pltpu.VMEM((_ROW_LEN,), jnp.int32),
            pltpu.VMEM((8, 128), jnp.int32),
            pltpu.VMEM((_CHUNK,), jnp.int32),
            pltpu.VMEM((_CHUNK + 16,), jnp.int32),
            pltpu.VMEM((256,), jnp.int32),
            pltpu.VMEM((_SEG,), jnp.int32),
            pltpu.VMEM((_VPT,), jnp.int32),
            pltpu.VMEM((_NCH, _K), jnp.int32),
            pltpu.VMEM((_NCH, _K), jnp.int32),
            pltpu.VMEM((16,), jnp.int32),
            pltpu.VMEM((16,), jnp.int32),
            pltpu.VMEM((16, _D), jnp.float32),
            pltpu.VMEM((_K, _D), jnp.float32),
            pltpu.VMEM((_K, _D), jnp.float32),
            pltpu.VMEM((_K, _D), jnp.float32),
            pltpu.VMEM_SHARED((2, _INVW), jnp.int32),
            pltpu.VMEM_SHARED((256,), jnp.int32),
            pltpu.SemaphoreType.DMA,
            pltpu.SemaphoreType.DMA,
            pltpu.SemaphoreType.DMA,
            pltpu.SemaphoreType.DMA,
            pltpu.SemaphoreType.DMA,
            pltpu.SemaphoreType.DMA,
            pltpu.SemaphoreType.DMA,
        ],
        compiler_params=pltpu.CompilerParams(needs_layout_passes=False,
                                             use_tc_tiling_on_sc=False),
    )(_sc_body)
    return call(flat_inp, table)


def kernel(input, table):
    b, l = input.shape
    out = _lpe(input.reshape(-1), table)
    return out.reshape(b, l, table.shape[1])


# speedup vs baseline: 1.0044x; 1.0044x over previous
import functools

import jax
import jax.numpy as jnp
from jax import lax
from jax.experimental import pallas as pl
from jax.experimental.pallas import tpu as pltpu
from jax.experimental.pallas import tpu_sc as plsc

_PAD = 1
_ROW_LEN = 8192
_D = 1024
_CHUNK_ELEMS = 1024
_K = 32
_NCH = _CHUNK_ELEMS // _K
_NBUF = 3


def _sc_body(inp_hbm, table_hbm, out_hbm, row_v, pos_v, buf0, buf1, buf2,
             carry_s, sg0, sg1, sg2, so0, so1, so2):
    c = lax.axis_index("c")
    s = lax.axis_index("s")
    chunk = c * 16 + s
    row = chunk // (_ROW_LEN // _CHUNK_ELEMS)
    p = chunk % (_ROW_LEN // _CHUNK_ELEMS)

    row_base = pl.multiple_of(row * _ROW_LEN, _ROW_LEN)
    pltpu.sync_copy(inp_hbm.at[pl.ds(row_base, _ROW_LEN)], row_v)

    def mask16(off):
        x = row_v[pl.ds(off, 16)]
        return jnp.minimum(jnp.abs(x - jnp.int32(_PAD)), jnp.int32(1))

    def acc_body(i, acc):
        for j in range(4):
            acc = acc + mask16(i * 64 + j * 16)
        return acc

    acc = lax.fori_loop(0, p * (_CHUNK_ELEMS // 64), acc_body,
                        jnp.zeros((16,), jnp.int32))
    carry_s[0] = jnp.sum(acc)

    chunk_base = p * _CHUNK_ELEMS

    def compute_piece(q):
        carry = carry_s[0]
        for j in range(2):
            m = mask16(chunk_base + q * _K + j * 16)
            cs = plsc.cumsum(m) + carry
            pos_v[pl.ds(q * _K + j * 16, 16)] = cs * m
            carry = carry + jnp.sum(m)
        carry_s[0] = carry

    out_base = chunk * _CHUNK_ELEMS
    bufs = (buf0, buf1, buf2)
    sgs = (sg0, sg1, sg2)
    sos = (so0, so1, so2)

    def start_gather(g, b):
        idx = pos_v.at[pl.ds(pl.multiple_of(g * _K, _K), _K)]
        pltpu.async_copy(table_hbm.at[idx], bufs[b], sgs[b])

    def wait_gather(b):
        pltpu.make_async_copy(table_hbm.at[pl.ds(0, _K)], bufs[b],
                              sgs[b]).wait()

    def start_write(g, b):
        dst = out_hbm.at[pl.ds(pl.multiple_of(out_base + g * _K, _K), _K)]
        pltpu.async_copy(bufs[b], dst, sos[b])

    def wait_write(b):
        pltpu.make_async_copy(bufs[b], out_hbm.at[pl.ds(0, _K)],
                              sos[b]).wait()

    compute_piece(0)
    start_gather(0, 0)
    compute_piece(1)
    start_gather(1, 1)
    compute_piece(2)
    wait_gather(0)
    start_write(0, 0)
    start_gather(2, 2)

    def pipe_body(i, carry):
        for b in range(_NBUF):
            g = _NBUF * i + 1 + b

            @pl.when(g + 2 < _NCH)
            def _():
                compute_piece(g + 2)

            wait_gather((1 + b) % _NBUF)
            start_write(g, (1 + b) % _NBUF)
            wait_write(b)

            @pl.when(g + 2 < _NCH)
            def _():
                start_gather(g + 2, b)
        return carry

    lax.fori_loop(0, (_NCH - 2) // _NBUF, pipe_body, 0)

    g_last = _NCH - 1
    wait_gather(g_last % _NBUF)
    start_write(g_last, g_last % _NBUF)
    wait_write((g_last - 1) % _NBUF)
    wait_write(g_last % _NBUF)


@jax.jit
def _lpe(flat_inp, table):
    n_tokens = flat_inp.shape[0]
    mesh = plsc.VectorSubcoreMesh(core_axis_name="c", subcore_axis_name="s")
    call = functools.partial(
        pl.kernel,
        mesh=mesh,
        out_type=jax.ShapeDtypeStruct((n_tokens, _D), jnp.float32),
        scratch_types=[
            pltpu.VMEM((_ROW_LEN,), jnp.int32),
            pltpu.VMEM((_CHUNK_ELEMS,), jnp.int32),
            pltpu.VMEM((_K, _D), jnp.float32),
            pltpu.VMEM((_K, _D), jnp.float32),
            pltpu.VMEM((_K, _D), jnp.float32),
            pltpu.SMEM((1,), jnp.int32),
            pltpu.SemaphoreType.DMA,
            pltpu.SemaphoreType.DMA,
            pltpu.SemaphoreType.DMA,
            pltpu.SemaphoreType.DMA,
            pltpu.SemaphoreType.DMA,
            pltpu.SemaphoreType.DMA,
        ],
        compiler_params=pltpu.CompilerParams(needs_layout_passes=False,
                                             use_tc_tiling_on_sc=False),
    )(_sc_body)
    return call(flat_inp, table)


def kernel(input, table):
    b, l = input.shape
    out = _lpe(input.reshape(-1), table)
    return out.reshape(b, l, table.shape[1])


# speedup vs baseline: 2.7854x; 2.7734x over previous
import functools

import jax
import jax.numpy as jnp
from jax import lax
from jax.experimental import pallas as pl
from jax.experimental.pallas import tpu as pltpu
from jax.experimental.pallas import tpu_sc as plsc

_PAD = 1
_ROW_LEN = 8192
_D = 1024
_CHUNK = 1024
_K = 32
_VPT = 512
_NCH = _VPT // _K
_INVW = 8704
_DUMW = 8400
_SEG = _INVW // 8
_BIG = 1 << 30


def _sc_body(inp_hbm, table_hbm, out_hbm,
             row_v, pos_v, val_v, padlist_v, fpall_v, seg_v, gidx_v,
             inv0_v, inv1_v, pidx_v, buf0, buf1, buf2,
             inv0_sh, inv1_sh, meta_sh,
             sg0, sg1, sg2, so0, so1, so2, msem):
    c = lax.axis_index("c")
    s = lax.axis_index("s")
    r_local = s // 8
    p = s % 8
    row_glob = 2 * c + r_local
    flat_row_base = row_glob * _ROW_LEN
    chunk_tok_base = flat_row_base + p * _CHUNK
    chunk_local = p * _CHUNK
    iota = lax.iota(jnp.int32, 16)

    pltpu.sync_copy(inp_hbm.at[pl.ds(pl.multiple_of(row_glob * _ROW_LEN,
                                                    _ROW_LEN), _ROW_LEN)],
                    row_v)

    def mask16(off):
        x = row_v[pl.ds(off, 16)]
        return jnp.minimum(jnp.abs(x - jnp.int32(_PAD)), jnp.int32(1))

    def acc_body(i, acc):
        for j in range(4):
            acc = acc + mask16(i * 64 + j * 16)
        return acc

    acc = lax.fori_loop(0, p * (_CHUNK // 64), acc_body,
                        jnp.zeros((16,), jnp.int32))
    carry = jnp.sum(acc)

    padcnt = jnp.int32(0)
    firstpad = _BIG
    for q in range(8):
        for j in range(8):
            t = q * 128 + j * 16
            m = mask16(chunk_local + t)
            cs = plsc.cumsum(m) + carry
            carry = carry + jnp.sum(m)
            iglob = chunk_tok_base + t + iota
            pos_v[2 * q, pl.ds(j * 16, 16)] = (
                cs * m - 1 + (1 - m) * jnp.int32(_DUMW + 1))
            val_v[pl.ds(t, 16)] = iglob
            pad = 1 - m
            ranks = plsc.cumsum(pad) + padcnt - 1
            lidx = ranks * pad + (1 - pad) * jnp.int32(_CHUNK)
            plsc.store_scatter(padlist_v, [lidx], iglob)
            padcnt = padcnt + jnp.sum(pad)
            firstpad = jnp.minimum(
                firstpad, jnp.min(iglob * pad + m * _BIG))

    pidx_v[pl.ds(0, 16)] = jnp.zeros((16,), jnp.int32) + firstpad
    pltpu.sync_copy(pidx_v, meta_sh.at[pl.ds(pl.multiple_of(s * 16, 16), 16)])
    plsc.subcore_barrier()
    pltpu.sync_copy(meta_sh, fpall_v)
    fpacc = jnp.zeros((16,), jnp.int32) + _BIG
    for t in range(8):
        fpacc = jnp.minimum(
            fpacc, fpall_v[pl.ds(r_local * 128 + t * 16, 16)])
    fp_use = jnp.minimum(jnp.min(fpacc), flat_row_base + _ROW_LEN - 1)

    fpuse_v = jnp.zeros((16,), jnp.int32) + fp_use

    def seg_body(i, cv):
        seg_v[pl.ds(i * 16, 16)] = fpuse_v
        return cv

    lax.fori_loop(0, _SEG // 16, seg_body, 0)

    @pl.when(r_local == 0)
    def _():
        pltpu.sync_copy(seg_v, inv0_sh.at[pl.ds(p * _SEG, _SEG)])

    @pl.when(r_local == 1)
    def _():
        pltpu.sync_copy(seg_v, inv1_sh.at[pl.ds(p * _SEG, _SEG)])

    plsc.subcore_barrier()

    def scatter_inv(inv_sh):
        for q in range(8):
            pltpu.async_copy(val_v.at[pl.ds(q * 128, 128)],
                             inv_sh.at[pos_v.at[2 * q]], msem)
        for q in range(8):
            pltpu.make_async_copy(val_v.at[pl.ds(0, 128)],
                                  inv_sh.at[pl.ds(0, 128)], msem).wait()

    @pl.when(r_local == 0)
    def _():
        scatter_inv(inv0_sh)

    @pl.when(r_local == 1)
    def _():
        scatter_inv(inv1_sh)

    plsc.subcore_barrier()

    wbase = s * _VPT
    for g in range(_NCH):
        pltpu.async_copy(inv0_sh.at[pl.ds(wbase + g * _K, _K)],
                         inv0_v.at[2 * g], msem)
        pltpu.async_copy(inv1_sh.at[pl.ds(wbase + g * _K, _K)],
                         inv1_v.at[2 * g], msem)
    for g in range(_NCH):
        pltpu.make_async_copy(inv0_sh.at[pl.ds(0, _K)], inv0_v.at[2 * g],
                              msem).wait()
        pltpu.make_async_copy(inv1_sh.at[pl.ds(0, _K)], inv1_v.at[2 * g],
                              msem).wait()
    for g in range(_NCH):
        base = wbase + g * _K + 1
        gidx_v[pl.ds(g * _K, 16)] = base + iota
        gidx_v[pl.ds(g * _K + 16, 16)] = base + 16 + iota

    bufs = (buf0, buf1, buf2)
    sgs = (sg0, sg1, sg2)
    sos = (so0, so1, so2)

    def start_g(g, b):
        idx = gidx_v.at[pl.ds(pl.multiple_of(g * _K, _K), _K)]
        pltpu.async_copy(table_hbm.at[idx], bufs[b], sgs[b])

    def wait_g(b):
        pltpu.make_async_copy(table_hbm.at[pl.ds(0, _K)], bufs[b],
                              sgs[b]).wait()

    def start_s(g, b):
        pltpu.async_copy(bufs[b], out_hbm.at[inv0_v.at[2 * g]], sos[b])
        pltpu.async_copy(bufs[b], out_hbm.at[inv1_v.at[2 * g]], sos[b])

    def wait_s(b):
        for _ in range(2):
            pltpu.make_async_copy(bufs[b], out_hbm.at[pl.ds(0, _K)],
                                  sos[b]).wait()

    start_g(0, 0)
    start_g(1, 1)
    wait_g(0)
    start_s(0, 0)
    start_g(2, 2)
    wait_g(1)
    start_s(1, 1)
    wait_s(0)
    start_g(3, 0)

    def ring(i, cv):
        for b in range(3):
            g = 3 * i + 2 + b
            bb = (2 + b) % 3
            wait_g(bb)
            start_s(g, bb)
            wait_s((1 + b) % 3)
            start_g(g + 2, (1 + b) % 3)
        return cv

    lax.fori_loop(0, (_NCH - 4) // 3, ring, 0)
    wait_g(2)
    start_s(14, 2)
    wait_s(1)
    wait_g(0)
    start_s(15, 0)
    wait_s(2)
    wait_s(0)
    plsc.subcore_barrier()

    def pad_body(k, cv):
        piece = padlist_v[pl.ds(pl.multiple_of((k // 16) * 16, 16), 16)]
        delta = 1 - jnp.minimum(jnp.abs(iota - k % 16), 1)
        flat = jnp.sum(piece * delta)
        pltpu.sync_copy(table_hbm.at[pl.ds(0, 1)],
                        out_hbm.at[pl.ds(flat, 1)])
        return cv

    lax.fori_loop(0, padcnt, pad_body, 0)


@jax.jit
def _lpe(flat_inp, table):
    n_tokens = flat_inp.shape[0]
    mesh = plsc.VectorSubcoreMesh(core_axis_name="c", subcore_axis_name="s")
    call = functools.partial(
        pl.kernel,
        mesh=mesh,
        out_type=jax.ShapeDtypeStruct((n_tokens, _D), jnp.float32),
        scratch_types=[
            pltpu.VMEM((_ROW_LEN,), jnp.int32),
            pltpu.VMEM((16, 128), jnp.int32),
            pltpu.VMEM((_CHUNK,), jnp.int32),
            pltpu.VMEM((_CHUNK + 16,), jnp.int32),
            pltpu.VMEM((256,), jnp.int32),
            pltpu.VMEM((_SEG,), jnp.int32),
            pltpu.VMEM((_VPT,), jnp.int32),
            pltpu.VMEM((2 * _NCH, _K), jnp.int32),
            pltpu.VMEM((2 * _NCH, _K), jnp.int32),
            pltpu.VMEM((16,), jnp.int32),
            pltpu.VMEM((_K, _D), jnp.float32),
            pltpu.VMEM((_K, _D), jnp.float32),
            pltpu.VMEM((_K, _D), jnp.float32),
            pltpu.VMEM_SHARED((_INVW,), jnp.int32),
            pltpu.VMEM_SHARED((_INVW,), jnp.int32),
            pltpu.VMEM_SHARED((256,), jnp.int32),
            pltpu.SemaphoreType.DMA,
            pltpu.SemaphoreType.DMA,
            pltpu.SemaphoreType.DMA,
            pltpu.SemaphoreType.DMA,
            pltpu.SemaphoreType.DMA,
            pltpu.SemaphoreType.DMA,
            pltpu.SemaphoreType.DMA,
        ],
        compiler_params=pltpu.CompilerParams(needs_layout_passes=False),
    )(_sc_body)
    return call(flat_inp, table)


def kernel(input, table):
    b, l = input.shape
    out = _lpe(input.reshape(-1), table)
    return out.reshape(b, l, table.shape[1])


# speedup vs baseline: 2.8404x; 1.0197x over previous
import functools

import jax
import jax.numpy as jnp
from jax import lax
from jax.experimental import pallas as pl
from jax.experimental.pallas import tpu as pltpu
from jax.experimental.pallas import tpu_sc as plsc

_PAD = 1
_ROW_LEN = 8192
_D = 1024
_CHUNK = 1024
_K = 32
_VPT = 512
_NCH = _VPT // _K
_INVW = 8704
_DUMW = 8400
_SEG = _INVW // 8
_BIG = 1 << 30


def _sc_body(inp_hbm, table_hbm, out_hbm,
             row_v, pos_v, val_v, padlist_v, fpall_v, seg_v, gidx_v,
             inv0_v, inv1_v, inv0l_v, inv1l_v, pidx_v, buf0, buf1, buf2,
             inv0_sh, inv1_sh, meta_sh,
             sg0, sg1, sg2, so0, so1, so2, msem):
    c = lax.axis_index("c")
    s = lax.axis_index("s")
    r_local = s // 8
    p = s % 8
    row_glob = 2 * c + r_local
    flat_row_base = row_glob * _ROW_LEN
    chunk_tok_base = flat_row_base + p * _CHUNK
    chunk_local = p * _CHUNK
    iota = lax.iota(jnp.int32, 16)
    wbase = s * _VPT
    bufs = (buf0, buf1, buf2)
    sgs = (sg0, sg1, sg2)
    sos = (so0, so1, so2)

    for g in range(_NCH):
        base = wbase + g * _K + 1
        gidx_v[pl.ds(g * _K, 16)] = base + iota
        gidx_v[pl.ds(g * _K + 16, 16)] = base + 16 + iota

    def start_g(g, b):
        idx = gidx_v.at[pl.ds(pl.multiple_of(g * _K, _K), _K)]
        pltpu.async_copy(table_hbm.at[idx], bufs[b], sgs[b])

    def wait_g(b):
        pltpu.make_async_copy(table_hbm.at[pl.ds(0, _K)], bufs[b],
                              sgs[b]).wait()

    def start_s(g, b):
        for inv_v, invl_v in ((inv0_v, inv0l_v), (inv1_v, inv1l_v)):
            a = invl_v[pl.ds(pl.multiple_of(g * _K, _K), 16)]
            b2 = invl_v[pl.ds(pl.multiple_of(g * _K, _K) + 16, 16)]
            v0 = jnp.sum(a * (1 - jnp.minimum(iota, 1)))
            ncontig = (jnp.sum(jnp.abs(a - (v0 + iota))) +
                       jnp.sum(jnp.abs(b2 - (v0 + 16 + iota))) +
                       v0 % 8)

            @pl.when(ncontig == 0)
            def _(inv_v=inv_v, v0=v0, b=b):
                pltpu.async_copy(bufs[b],
                                 out_hbm.at[pl.ds(pl.multiple_of(v0, 8),
                                                  _K)], sos[b])

            @pl.when(ncontig != 0)
            def _(inv_v=inv_v, b=b):
                pltpu.async_copy(bufs[b], out_hbm.at[inv_v.at[2 * g]],
                                 sos[b])

    def wait_s(b):
        for _ in range(2):
            pltpu.make_async_copy(bufs[b], out_hbm.at[pl.ds(0, _K)],
                                  sos[b]).wait()

    start_g(0, 0)
    start_g(1, 1)
    start_g(2, 2)

    pltpu.sync_copy(inp_hbm.at[pl.ds(pl.multiple_of(row_glob * _ROW_LEN,
                                                    _ROW_LEN), _ROW_LEN)],
                    row_v)

    def mask16(off):
        x = row_v[pl.ds(off, 16)]
        return jnp.minimum(jnp.abs(x - jnp.int32(_PAD)), jnp.int32(1))

    def acc_body(i, acc):
        for j in range(4):
            acc = acc + mask16(i * 64 + j * 16)
        return acc

    acc = lax.fori_loop(0, p * (_CHUNK // 64), acc_body,
                        jnp.zeros((16,), jnp.int32))
    carry = jnp.sum(acc)

    padcnt = jnp.int32(0)
    firstpad = _BIG
    for q in range(8):
        for j in range(8):
            t = q * 128 + j * 16
            m = mask16(chunk_local + t)
            cs = plsc.cumsum(m) + carry
            carry = carry + jnp.sum(m)
            iglob = chunk_tok_base + t + iota
            pos_v[2 * q, pl.ds(j * 16, 16)] = (
                cs * m - 1 + (1 - m) * jnp.int32(_DUMW + 1))
            val_v[pl.ds(t, 16)] = iglob
            pad = 1 - m
            ranks = plsc.cumsum(pad) + padcnt - 1
            lidx = ranks * pad + (1 - pad) * jnp.int32(_CHUNK)
            plsc.store_scatter(padlist_v, [lidx], iglob)
            padcnt = padcnt + jnp.sum(pad)
            firstpad = jnp.minimum(
                firstpad, jnp.min(iglob * pad + m * _BIG))

    pidx_v[pl.ds(0, 16)] = jnp.zeros((16,), jnp.int32) + firstpad
    pltpu.sync_copy(pidx_v, meta_sh.at[pl.ds(pl.multiple_of(s * 16, 16), 16)])
    plsc.subcore_barrier()
    pltpu.sync_copy(meta_sh, fpall_v)
    fpacc = jnp.zeros((16,), jnp.int32) + _BIG
    for t in range(8):
        fpacc = jnp.minimum(
            fpacc, fpall_v[pl.ds(r_local * 128 + t * 16, 16)])
    fp_use = jnp.minimum(jnp.min(fpacc), flat_row_base + _ROW_LEN - 1)

    fpuse_v = jnp.zeros((16,), jnp.int32) + fp_use

    def seg_body(i, cv):
        seg_v[pl.ds(i * 16, 16)] = fpuse_v
        return cv

    lax.fori_loop(0, _SEG // 16, seg_body, 0)

    @pl.when(r_local == 0)
    def _():
        pltpu.sync_copy(seg_v, inv0_sh.at[pl.ds(p * _SEG, _SEG)])

    @pl.when(r_local == 1)
    def _():
        pltpu.sync_copy(seg_v, inv1_sh.at[pl.ds(p * _SEG, _SEG)])

    plsc.subcore_barrier()

    def scatter_inv(inv_sh):
        for q in range(8):
            pltpu.async_copy(val_v.at[pl.ds(q * 128, 128)],
                             inv_sh.at[pos_v.at[2 * q]], msem)
        for q in range(8):
            pltpu.make_async_copy(val_v.at[pl.ds(0, 128)],
                                  inv_sh.at[pl.ds(0, 128)], msem).wait()

    @pl.when(r_local == 0)
    def _():
        scatter_inv(inv0_sh)

    @pl.when(r_local == 1)
    def _():
        scatter_inv(inv1_sh)

    plsc.subcore_barrier()

    for g in range(_NCH):
        pltpu.async_copy(inv0_sh.at[pl.ds(wbase + g * _K, _K)],
                         inv0_v.at[2 * g], msem)
        pltpu.async_copy(inv1_sh.at[pl.ds(wbase + g * _K, _K)],
                         inv1_v.at[2 * g], msem)
    pltpu.async_copy(inv0_sh.at[pl.ds(wbase, _VPT)], inv0l_v, msem)
    pltpu.async_copy(inv1_sh.at[pl.ds(wbase, _VPT)], inv1l_v, msem)
    for g in range(_NCH):
        pltpu.make_async_copy(inv0_sh.at[pl.ds(0, _K)], inv0_v.at[2 * g],
                              msem).wait()
        pltpu.make_async_copy(inv1_sh.at[pl.ds(0, _K)], inv1_v.at[2 * g],
                              msem).wait()
    pltpu.make_async_copy(inv0_sh.at[pl.ds(0, _VPT)], inv0l_v, msem).wait()
    pltpu.make_async_copy(inv1_sh.at[pl.ds(0, _VPT)], inv1l_v, msem).wait()
    wait_g(0)
    start_s(0, 0)
    wait_g(1)
    start_s(1, 1)
    wait_s(0)
    start_g(3, 0)

    def ring(i, cv):
        for b in range(3):
            g = 3 * i + 2 + b
            bb = (2 + b) % 3
            wait_g(bb)
            start_s(g, bb)
            wait_s((1 + b) % 3)
            start_g(g + 2, (1 + b) % 3)
        return cv

    lax.fori_loop(0, (_NCH - 4) // 3, ring, 0)
    wait_g(2)
    start_s(14, 2)
    wait_s(1)
    wait_g(0)
    start_s(15, 0)
    wait_s(2)
    wait_s(0)
    plsc.subcore_barrier()

    def pad_body(k, cv):
        piece = padlist_v[pl.ds(pl.multiple_of((k // 16) * 16, 16), 16)]
        delta = 1 - jnp.minimum(jnp.abs(iota - k % 16), 1)
        flat = jnp.sum(piece * delta)
        pltpu.sync_copy(table_hbm.at[pl.ds(0, 1)],
                        out_hbm.at[pl.ds(flat, 1)])
        return cv

    lax.fori_loop(0, padcnt, pad_body, 0)


@jax.jit
def _lpe(flat_inp, table):
    n_tokens = flat_inp.shape[0]
    mesh = plsc.VectorSubcoreMesh(core_axis_name="c", subcore_axis_name="s")
    call = functools.partial(
        pl.kernel,
        mesh=mesh,
        out_type=jax.ShapeDtypeStruct((n_tokens, _D), jnp.float32),
        scratch_types=[
            pltpu.VMEM((_ROW_LEN,), jnp.int32),
            pltpu.VMEM((16, 128), jnp.int32),
            pltpu.VMEM((_CHUNK,), jnp.int32),
            pltpu.VMEM((_CHUNK + 16,), jnp.int32),
            pltpu.VMEM((256,), jnp.int32),
            pltpu.VMEM((_SEG,), jnp.int32),
            pltpu.VMEM((_VPT,), jnp.int32),
            pltpu.VMEM((2 * _NCH, _K), jnp.int32),
            pltpu.VMEM((2 * _NCH, _K), jnp.int32),
            pltpu.VMEM((_VPT,), jnp.int32),
            pltpu.VMEM((_VPT,), jnp.int32),
            pltpu.VMEM((16,), jnp.int32),
            pltpu.VMEM((_K, _D), jnp.float32),
            pltpu.VMEM((_K, _D), jnp.float32),
            pltpu.VMEM((_K, _D), jnp.float32),
            pltpu.VMEM_SHARED((_INVW,), jnp.int32),
            pltpu.VMEM_SHARED((_INVW,), jnp.int32),
            pltpu.VMEM_SHARED((256,), jnp.int32),
            pltpu.SemaphoreType.DMA,
            pltpu.SemaphoreType.DMA,
            pltpu.SemaphoreType.DMA,
            pltpu.SemaphoreType.DMA,
            pltpu.SemaphoreType.DMA,
            pltpu.SemaphoreType.DMA,
            pltpu.SemaphoreType.DMA,
        ],
        compiler_params=pltpu.CompilerParams(needs_layout_passes=False),
    )(_sc_body)
    return call(flat_inp, table)


def kernel(input, table):
    b, l = input.shape
    out = _lpe(input.reshape(-1), table)
    return out.reshape(b, l, table.shape[1])


# speedup vs baseline: 2.8527x; 1.0043x over previous
import functools

import jax
import jax.numpy as jnp
from jax import lax
from jax.experimental import pallas as pl
from jax.experimental.pallas import tpu as pltpu
from jax.experimental.pallas import tpu_sc as plsc

_PAD = 1
_ROW_LEN = 8192
_D = 1024
_CHUNK = 1024
_K = 32
_VPT = 512
_NCH = _VPT // _K
_INVW = 8704
_DUMW = 8400
_SEG = _INVW // 8
_BIG = 1 << 30


def _sc_body(inp_hbm, table_hbm, out_hbm,
             row_v, pos_v, val_v, padlist_v, fpall_v, seg_v, gidx_v,
             inv0l_v, inv1l_v, pidx_v, buf0, buf1, buf2,
             inv0_sh, inv1_sh, meta_sh,
             sg0, sg1, sg2, so0, so1, so2, msem):
    c = lax.axis_index("c")
    s = lax.axis_index("s")
    r_local = s // 8
    p = s % 8
    row_glob = 2 * c + r_local
    flat_row_base = row_glob * _ROW_LEN
    chunk_tok_base = flat_row_base + p * _CHUNK
    chunk_local = p * _CHUNK
    iota = lax.iota(jnp.int32, 16)
    wbase = s * _VPT
    bufs = (buf0, buf1, buf2)
    sgs = (sg0, sg1, sg2)
    sos = (so0, so1, so2)

    for g in range(_NCH):
        base = wbase + g * _K + 1
        gidx_v[pl.ds(g * _K, 16)] = base + iota
        gidx_v[pl.ds(g * _K + 16, 16)] = base + 16 + iota

    def start_g(g, b):
        idx = gidx_v.at[pl.ds(pl.multiple_of(g * _K, _K), _K)]
        pltpu.async_copy(table_hbm.at[idx], bufs[b], sgs[b])

    def wait_g(b):
        pltpu.make_async_copy(table_hbm.at[pl.ds(0, _K)], bufs[b],
                              sgs[b]).wait()

    def start_s(g, b):
        for invl_v in (inv0l_v, inv1l_v):
            a = invl_v[pl.ds(pl.multiple_of(g * _K, _K), 16)]
            b2 = invl_v[pl.ds(pl.multiple_of(g * _K, _K) + 16, 16)]
            v0 = jnp.sum(a * (1 - jnp.minimum(iota, 1)))
            ncontig = (jnp.sum(jnp.abs(a - (v0 + iota))) +
                       jnp.sum(jnp.abs(b2 - (v0 + 16 + iota))) +
                       v0 % 8)

            @pl.when(ncontig == 0)
            def _(v0=v0, b=b):
                pltpu.async_copy(bufs[b],
                                 out_hbm.at[pl.ds(pl.multiple_of(v0, 8),
                                                  _K)], sos[b])

            @pl.when(ncontig != 0)
            def _(invl_v=invl_v, b=b):
                idx = invl_v.at[pl.ds(pl.multiple_of(g * _K, _K), _K)]
                pltpu.async_copy(bufs[b], out_hbm.at[idx], sos[b])

    def wait_s(b):
        for _ in range(2):
            pltpu.make_async_copy(bufs[b], out_hbm.at[pl.ds(0, _K)],
                                  sos[b]).wait()

    start_g(0, 0)
    start_g(1, 1)
    start_g(2, 2)

    pltpu.sync_copy(inp_hbm.at[pl.ds(pl.multiple_of(row_glob * _ROW_LEN,
                                                    _ROW_LEN), _ROW_LEN)],
                    row_v)

    def mask16(off):
        x = row_v[pl.ds(off, 16)]
        return jnp.minimum(jnp.abs(x - jnp.int32(_PAD)), jnp.int32(1))

    def acc_body(i, acc):
        for j in range(4):
            acc = acc + mask16(i * 64 + j * 16)
        return acc

    acc = lax.fori_loop(0, p * (_CHUNK // 64), acc_body,
                        jnp.zeros((16,), jnp.int32))
    carry = jnp.sum(acc)

    padcnt = jnp.int32(0)
    firstpad = _BIG
    for q in range(8):
        for j in range(8):
            t = q * 128 + j * 16
            m = mask16(chunk_local + t)
            cs = plsc.cumsum(m) + carry
            carry = carry + jnp.sum(m)
            iglob = chunk_tok_base + t + iota
            pos_v[2 * q, pl.ds(j * 16, 16)] = (
                cs * m - 1 + (1 - m) * jnp.int32(_DUMW + 1))
            val_v[pl.ds(t, 16)] = iglob
            pad = 1 - m
            ranks = plsc.cumsum(pad) + padcnt - 1
            lidx = ranks * pad + (1 - pad) * jnp.int32(_CHUNK)
            plsc.store_scatter(padlist_v, [lidx], iglob)
            padcnt = padcnt + jnp.sum(pad)
            firstpad = jnp.minimum(
                firstpad, jnp.min(iglob * pad + m * _BIG))

    pidx_v[pl.ds(0, 16)] = jnp.zeros((16,), jnp.int32) + firstpad
    pltpu.sync_copy(pidx_v, meta_sh.at[pl.ds(pl.multiple_of(s * 16, 16), 16)])
    plsc.subcore_barrier()
    pltpu.sync_copy(meta_sh, fpall_v)
    fpacc = jnp.zeros((16,), jnp.int32) + _BIG
    for t in range(8):
        fpacc = jnp.minimum(
            fpacc, fpall_v[pl.ds(r_local * 128 + t * 16, 16)])
    fp_use = jnp.minimum(jnp.min(fpacc), flat_row_base + _ROW_LEN - 1)

    fpuse_v = jnp.zeros((16,), jnp.int32) + fp_use

    def seg_body(i, cv):
        seg_v[pl.ds(i * 16, 16)] = fpuse_v
        return cv

    lax.fori_loop(0, _SEG // 16, seg_body, 0)

    @pl.when(r_local == 0)
    def _():
        pltpu.sync_copy(seg_v, inv0_sh.at[pl.ds(p * _SEG, _SEG)])

    @pl.when(r_local == 1)
    def _():
        pltpu.sync_copy(seg_v, inv1_sh.at[pl.ds(p * _SEG, _SEG)])

    plsc.subcore_barrier()

    def scatter_inv(inv_sh):
        for q in range(8):
            pltpu.async_copy(val_v.at[pl.ds(q * 128, 128)],
                             inv_sh.at[pos_v.at[2 * q]], msem)
        for q in range(8):
            pltpu.make_async_copy(val_v.at[pl.ds(0, 128)],
                                  inv_sh.at[pl.ds(0, 128)], msem).wait()

    @pl.when(r_local == 0)
    def _():
        scatter_inv(inv0_sh)

    @pl.when(r_local == 1)
    def _():
        scatter_inv(inv1_sh)

    plsc.subcore_barrier()

    pltpu.async_copy(inv0_sh.at[pl.ds(wbase, _VPT)], inv0l_v, msem)
    pltpu.async_copy(inv1_sh.at[pl.ds(wbase, _VPT)], inv1l_v, msem)
    pltpu.make_async_copy(inv0_sh.at[pl.ds(0, _VPT)], inv0l_v, msem).wait()
    pltpu.make_async_copy(inv1_sh.at[pl.ds(0, _VPT)], inv1l_v, msem).wait()
    wait_g(0)
    start_s(0, 0)
    wait_g(1)
    start_s(1, 1)
    wait_s(0)
    start_g(3, 0)

    def ring(i, cv):
        for b in range(3):
            g = 3 * i + 2 + b
            bb = (2 + b) % 3
            wait_g(bb)
            start_s(g, bb)
            wait_s((1 + b) % 3)
            start_g(g + 2, (1 + b) % 3)
        return cv

    lax.fori_loop(0, (_NCH - 4) // 3, ring, 0)
    wait_g(2)
    start_s(14, 2)
    wait_s(1)
    wait_g(0)
    start_s(15, 0)
    wait_s(2)
    wait_s(0)
    plsc.subcore_barrier()

    def pad_body(k, cv):
        piece = padlist_v[pl.ds(pl.multiple_of((k // 16) * 16, 16), 16)]
        delta = 1 - jnp.minimum(jnp.abs(iota - k % 16), 1)
        flat = jnp.sum(piece * delta)
        pltpu.sync_copy(table_hbm.at[pl.ds(0, 1)],
                        out_hbm.at[pl.ds(flat, 1)])
        return cv

    lax.fori_loop(0, padcnt, pad_body, 0)


@jax.jit
def _lpe(flat_inp, table):
    n_tokens = flat_inp.shape[0]
    mesh = plsc.VectorSubcoreMesh(core_axis_name="c", subcore_axis_name="s")
    call = functools.partial(
        pl.kernel,
        mesh=mesh,
        out_type=jax.ShapeDtypeStruct((n_tokens, _D), jnp.float32),
        scratch_types=[
            pltpu.VMEM((_ROW_LEN,), jnp.int32),
            pltpu.VMEM((16, 128), jnp.int32),
            pltpu.VMEM((_CHUNK,), jnp.int32),
            pltpu.VMEM((_CHUNK + 16,), jnp.int32),
            pltpu.VMEM((256,), jnp.int32),
            pltpu.VMEM((_SEG,), jnp.int32),
            pltpu.VMEM((_VPT,), jnp.int32),
            pltpu.VMEM((_VPT,), jnp.int32),
            pltpu.VMEM((_VPT,), jnp.int32),
            pltpu.VMEM((16,), jnp.int32),
            pltpu.VMEM((_K, _D), jnp.float32),
            pltpu.VMEM((_K, _D), jnp.float32),
            pltpu.VMEM((_K, _D), jnp.float32),
            pltpu.VMEM_SHARED((_INVW,), jnp.int32),
            pltpu.VMEM_SHARED((_INVW,), jnp.int32),
            pltpu.VMEM_SHARED((256,), jnp.int32),
            pltpu.SemaphoreType.DMA,
            pltpu.SemaphoreType.DMA,
            pltpu.SemaphoreType.DMA,
            pltpu.SemaphoreType.DMA,
            pltpu.SemaphoreType.DMA,
            pltpu.SemaphoreType.DMA,
            pltpu.SemaphoreType.DMA,
        ],
        compiler_params=pltpu.CompilerParams(needs_layout_passes=False),
    )(_sc_body)
    return call(flat_inp, table)


def kernel(input, table):
    b, l = input.shape
    out = _lpe(input.reshape(-1), table)
    return out.reshape(b, l, table.shape[1])


# speedup vs baseline: 2.8641x; 1.0040x over previous
import functools

import jax
import jax.numpy as jnp
from jax import lax
from jax.experimental import pallas as pl
from jax.experimental.pallas import tpu as pltpu
from jax.experimental.pallas import tpu_sc as plsc

_PAD = 1
_ROW_LEN = 8192
_D = 1024
_CHUNK = 1024
_K = 32
_VPT = 512
_NCH = _VPT // _K
_INVW = 8704
_DUMW = 8400
_SEG = _INVW // 8
_BIG = 1 << 30


def _sc_body(inp_hbm, table_hbm, out_hbm,
             row_v, pos_v, val_v, padlist_v, fpall_v, seg_v, gidx_v,
             inv0l_v, inv1l_v, pidx_v, buf0, buf1, buf2,
             inv0_sh, inv1_sh, meta_sh,
             sg0, sg1, sg2, so0, so1, so2, msem):
    c = lax.axis_index("c")
    s = lax.axis_index("s")
    r_local = s // 8
    p = s % 8
    row_glob = 2 * c + r_local
    flat_row_base = row_glob * _ROW_LEN
    chunk_tok_base = flat_row_base + p * _CHUNK
    chunk_local = p * _CHUNK
    iota = lax.iota(jnp.int32, 16)
    wbase = s * _VPT
    bufs = (buf0, buf1, buf2)
    sgs = (sg0, sg1, sg2)
    sos = (so0, so1, so2)

    for g in range(_NCH):
        base = wbase + g * _K + 1
        gidx_v[pl.ds(g * _K, 16)] = base + iota
        gidx_v[pl.ds(g * _K + 16, 16)] = base + 16 + iota

    def start_g(g, b):
        idx = gidx_v.at[pl.ds(pl.multiple_of(g * _K, _K), _K)]
        pltpu.async_copy(table_hbm.at[idx], bufs[b], sgs[b])

    def wait_g(b):
        pltpu.make_async_copy(table_hbm.at[pl.ds(0, _K)], bufs[b],
                              sgs[b]).wait()

    def start_s(g, b):
        for invl_v in (inv0l_v, inv1l_v):
            a = invl_v[pl.ds(pl.multiple_of(g * _K, _K), 16)]
            b2 = invl_v[pl.ds(pl.multiple_of(g * _K, _K) + 16, 16)]
            v0 = jnp.sum(a * (1 - jnp.minimum(iota, 1)))
            ncontig = (jnp.sum(jnp.abs(a - (v0 + iota))) +
                       jnp.sum(jnp.abs(b2 - (v0 + 16 + iota))) +
                       v0 % 8)

            @pl.when(ncontig == 0)
            def _(v0=v0, b=b):
                pltpu.async_copy(bufs[b],
                                 out_hbm.at[pl.ds(pl.multiple_of(v0, 8),
                                                  _K)], sos[b])

            @pl.when(ncontig != 0)
            def _(invl_v=invl_v, b=b):
                idx = invl_v.at[pl.ds(pl.multiple_of(g * _K, _K), _K)]
                pltpu.async_copy(bufs[b], out_hbm.at[idx], sos[b])

    def wait_s(b):
        for _ in range(2):
            pltpu.make_async_copy(bufs[b], out_hbm.at[pl.ds(0, _K)],
                                  sos[b]).wait()

    pltpu.sync_copy(inp_hbm.at[pl.ds(pl.multiple_of(row_glob * _ROW_LEN,
                                                    _ROW_LEN), _ROW_LEN)],
                    row_v)
    start_g(0, 0)
    start_g(1, 1)
    start_g(2, 2)

    def mask16(off):
        x = row_v[pl.ds(off, 16)]
        return jnp.minimum(jnp.abs(x - jnp.int32(_PAD)), jnp.int32(1))

    def acc_body(i, acc):
        for j in range(4):
            acc = acc + mask16(i * 64 + j * 16)
        return acc

    acc = lax.fori_loop(0, p * (_CHUNK // 64), acc_body,
                        jnp.zeros((16,), jnp.int32))
    carry = jnp.sum(acc)

    padcnt = jnp.int32(0)
    firstpad = _BIG
    for q in range(8):
        for j in range(8):
            t = q * 128 + j * 16
            m = mask16(chunk_local + t)
            cs = plsc.cumsum(m) + carry
            carry = carry + jnp.sum(m)
            iglob = chunk_tok_base + t + iota
            pos_v[2 * q, pl.ds(j * 16, 16)] = (
                cs * m - 1 + (1 - m) * jnp.int32(_DUMW + 1))
            val_v[pl.ds(t, 16)] = iglob
            pad = 1 - m
            ranks = plsc.cumsum(pad) + padcnt - 1
            lidx = ranks * pad + (1 - pad) * jnp.int32(_CHUNK)
            plsc.store_scatter(padlist_v, [lidx], iglob)
            padcnt = padcnt + jnp.sum(pad)
            firstpad = jnp.minimum(
                firstpad, jnp.min(iglob * pad + m * _BIG))

    pidx_v[pl.ds(0, 16)] = jnp.zeros((16,), jnp.int32) + firstpad
    pltpu.sync_copy(pidx_v, meta_sh.at[pl.ds(pl.multiple_of(s * 16, 16), 16)])
    plsc.subcore_barrier()
    pltpu.sync_copy(meta_sh, fpall_v)
    fpacc = jnp.zeros((16,), jnp.int32) + _BIG
    for t in range(8):
        fpacc = jnp.minimum(
            fpacc, fpall_v[pl.ds(r_local * 128 + t * 16, 16)])
    fp_use = jnp.minimum(jnp.min(fpacc), flat_row_base + _ROW_LEN - 1)

    fpuse_v = jnp.zeros((16,), jnp.int32) + fp_use

    def seg_body(i, cv):
        seg_v[pl.ds(i * 16, 16)] = fpuse_v
        return cv

    lax.fori_loop(0, _SEG // 16, seg_body, 0)

    @pl.when(r_local == 0)
    def _():
        pltpu.sync_copy(seg_v, inv0_sh.at[pl.ds(p * _SEG, _SEG)])

    @pl.when(r_local == 1)
    def _():
        pltpu.sync_copy(seg_v, inv1_sh.at[pl.ds(p * _SEG, _SEG)])

    plsc.subcore_barrier()

    def scatter_inv(inv_sh):
        for q in range(8):
            pltpu.async_copy(val_v.at[pl.ds(q * 128, 128)],
                             inv_sh.at[pos_v.at[2 * q]], msem)
        for q in range(8):
            pltpu.make_async_copy(val_v.at[pl.ds(0, 128)],
                                  inv_sh.at[pl.ds(0, 128)], msem).wait()

    @pl.when(r_local == 0)
    def _():
        scatter_inv(inv0_sh)

    @pl.when(r_local == 1)
    def _():
        scatter_inv(inv1_sh)

    plsc.subcore_barrier()

    pltpu.async_copy(inv0_sh.at[pl.ds(wbase, _VPT)], inv0l_v, msem)
    pltpu.async_copy(inv1_sh.at[pl.ds(wbase, _VPT)], inv1l_v, msem)
    pltpu.make_async_copy(inv0_sh.at[pl.ds(0, _VPT)], inv0l_v, msem).wait()
    pltpu.make_async_copy(inv1_sh.at[pl.ds(0, _VPT)], inv1l_v, msem).wait()
    wait_g(0)
    start_s(0, 0)
    wait_g(1)
    start_s(1, 1)
    wait_s(0)
    start_g(3, 0)

    def ring(i, cv):
        for b in range(3):
            g = 3 * i + 2 + b
            bb = (2 + b) % 3
            wait_g(bb)
            start_s(g, bb)
            wait_s((1 + b) % 3)
            start_g(g + 2, (1 + b) % 3)
        return cv

    lax.fori_loop(0, (_NCH - 4) // 3, ring, 0)
    wait_g(2)
    start_s(14, 2)
    wait_s(1)
    wait_g(0)
    start_s(15, 0)
    wait_s(2)
    wait_s(0)
    plsc.subcore_barrier()

    def pad_body(k, cv):
        piece = padlist_v[pl.ds(pl.multiple_of((k // 16) * 16, 16), 16)]
        delta = 1 - jnp.minimum(jnp.abs(iota - k % 16), 1)
        flat = jnp.sum(piece * delta)
        pltpu.sync_copy(table_hbm.at[pl.ds(0, 1)],
                        out_hbm.at[pl.ds(flat, 1)])
        return cv

    lax.fori_loop(0, padcnt, pad_body, 0)


@jax.jit
def _lpe(flat_inp, table):
    n_tokens = flat_inp.shape[0]
    mesh = plsc.VectorSubcoreMesh(core_axis_name="c", subcore_axis_name="s")
    call = functools.partial(
        pl.kernel,
        mesh=mesh,
        out_type=jax.ShapeDtypeStruct((n_tokens, _D), jnp.float32),
        scratch_types=[
            pltpu.VMEM((_ROW_LEN,), jnp.int32),
            pltpu.VMEM((16, 128), jnp.int32),
            pltpu.VMEM((_CHUNK,), jnp.int32),
            pltpu.VMEM((_CHUNK + 16,), jnp.int32),
            pltpu.VMEM((256,), jnp.int32),
            pltpu.VMEM((_SEG,), jnp.int32),
            pltpu.VMEM((_VPT,), jnp.int32),
            pltpu.VMEM((_VPT,), jnp.int32),
            pltpu.VMEM((_VPT,), jnp.int32),
            pltpu.VMEM((16,), jnp.int32),
            pltpu.VMEM((_K, _D), jnp.float32),
            pltpu.VMEM((_K, _D), jnp.float32),
            pltpu.VMEM((_K, _D), jnp.float32),
            pltpu.VMEM_SHARED((_INVW,), jnp.int32),
            pltpu.VMEM_SHARED((_INVW,), jnp.int32),
            pltpu.VMEM_SHARED((256,), jnp.int32),
            pltpu.SemaphoreType.DMA,
            pltpu.SemaphoreType.DMA,
            pltpu.SemaphoreType.DMA,
            pltpu.SemaphoreType.DMA,
            pltpu.SemaphoreType.DMA,
            pltpu.SemaphoreType.DMA,
            pltpu.SemaphoreType.DMA,
        ],
        compiler_params=pltpu.CompilerParams(needs_layout_passes=False),
    )(_sc_body)
    return call(flat_inp, table)


def kernel(input, table):
    b, l = input.shape
    out = _lpe(input.reshape(-1), table)
    return out.reshape(b, l, table.shape[1])
